# Initial kernel scaffold; baseline (speedup 1.0000x reference)
import jax
import jax.numpy as jnp
from jax.experimental import pallas as pl


def kernel(er_input, seg_label, gt_boundary_seg):
    raise NotImplementedError("write your pallas kernel here")


# speedup vs baseline: 14.5088x; 14.5088x over previous
import jax
import jax.numpy as jnp
from jax.experimental import pallas as pl
from jax.experimental.pallas import tpu as pltpu

_EPS = 1e-8
_H = 128
_W = 128
_C = 256
_B = 4
_C_CHUNK = 64
_NUM_CC = _C // _C_CHUNK

_SHIFTS = [(0, 1), (0, 2)] + [(di, dj) for di in (1, 2) for dj in range(-2, 3)]


def _roll2(x, di, dj):
    if di:
        x = jnp.roll(x, -di, axis=x.ndim - 2)
    if dj:
        x = jnp.roll(x, -dj, axis=x.ndim - 1)
    return x


def _cbl_kernel(seg_ref, gtb_ref, f_ref, out_ref, d_acc, n_acc, s_acc):
    b = pl.program_id(0)
    cc = pl.program_id(1)

    @pl.when(cc == 0)
    def _reset_image_acc():
        d_acc[...] = jnp.zeros_like(d_acc)
        n_acc[...] = jnp.zeros_like(n_acc)

    @pl.when(jnp.logical_and(b == 0, cc == 0))
    def _reset_global_acc():
        s_acc[0] = jnp.float32(0.0)
        s_acc[1] = jnp.float32(0.0)

    f = f_ref[0]
    n_acc[...] += jnp.sum(f * f, axis=0)
    for s_idx, (di, dj) in enumerate(_SHIFTS):
        fr = _roll2(f, di, dj)
        d_acc[s_idx] += jnp.sum(f * fr, axis=0)

    @pl.when(cc == _NUM_CC - 1)
    def _finalize_image():
        seg = seg_ref[0]
        gtb = gtb_ref[0]
        seg = jnp.where(seg == 255, 0, seg)
        gtb = jnp.where(gtb == 255, 0, gtb)
        r = jax.lax.broadcasted_iota(jnp.int32, (_H, _W), 0)
        c = jax.lax.broadcasted_iota(jnp.int32, (_H, _W), 1)
        interior = (r >= 2) & (r <= _H - 3) & (c >= 2) & (c <= _W - 3)
        maskf = jnp.where((gtb * seg > 0) & interior,
                          jnp.float32(1.0), jnp.float32(0.0))
        inv = jnp.float32(1.0) / jnp.maximum(jnp.sqrt(n_acc[...]),
                                             jnp.float32(_EPS))
        s_total = jnp.float32(0.0)
        for s_idx, (di, dj) in enumerate(_SHIFTS):
            inv_nb = _roll2(inv, di, dj)
            sim = d_acc[s_idx] * inv * inv_nb
            seg_nb = _roll2(seg, di, dj)
            slab = jnp.where(seg == seg_nb, jnp.float32(1.0), jnp.float32(0.0))
            w = maskf + _roll2(maskf, di, dj)
            s_total += jnp.sum(w * (sim - slab) ** 2)
        count = jnp.sum(maskf)
        valid = count >= jnp.float32(1.0)
        contrib = jnp.where(
            valid,
            s_total / (jnp.maximum(count, jnp.float32(1.0)) * jnp.float32(24.0)),
            jnp.float32(0.0))
        s_acc[0] += contrib
        s_acc[1] += jnp.where(valid, jnp.float32(1.0), jnp.float32(0.0))

        @pl.when(b == _B - 1)
        def _finalize_total():
            tot = s_acc[0] / jnp.maximum(s_acc[1], jnp.float32(1.0))
            tot = jnp.where(s_acc[1] == jnp.float32(0.0), jnp.float32(0.0), tot)
            tot = jnp.where(jnp.isnan(tot), jnp.float32(0.0), tot)
            out_ref[...] = jnp.full((1, 1), tot, dtype=jnp.float32)


def kernel(er_input, seg_label, gt_boundary_seg):
    seg_ds = seg_label[:, ::4, ::4]
    gtb_ds = gt_boundary_seg[:, ::4, ::4]

    out = pl.pallas_call(
        _cbl_kernel,
        grid=(_B, _NUM_CC),
        in_specs=[
            pl.BlockSpec((1, _H, _W), lambda b, cc: (b, 0, 0)),
            pl.BlockSpec((1, _H, _W), lambda b, cc: (b, 0, 0)),
            pl.BlockSpec((1, _C_CHUNK, _H, _W), lambda b, cc: (b, cc, 0, 0)),
        ],
        out_specs=pl.BlockSpec((1, 1), lambda b, cc: (0, 0)),
        out_shape=jax.ShapeDtypeStruct((1, 1), jnp.float32),
        scratch_shapes=[
            pltpu.VMEM((len(_SHIFTS), _H, _W), jnp.float32),
            pltpu.VMEM((_H, _W), jnp.float32),
            pltpu.SMEM((2,), jnp.float32),
        ],
    )(seg_ds, gtb_ds, er_input)
    return out.reshape(())


# speedup vs baseline: 14.5122x; 1.0002x over previous
import jax
import jax.numpy as jnp
from jax.experimental import pallas as pl
from jax.experimental.pallas import tpu as pltpu

_EPS = 1e-8
_H = 128
_W = 128
_C = 256
_B = 4
_C_CHUNK = 64
_NUM_CC = _C // _C_CHUNK

_SHIFTS = [(0, 1), (0, 2)] + [(di, dj) for di in (1, 2) for dj in range(-2, 3)]


def _roll2(x, di, dj):
    if di:
        x = jnp.roll(x, -di, axis=x.ndim - 2)
    if dj:
        x = jnp.roll(x, -dj, axis=x.ndim - 1)
    return x


def _cbl_kernel(seg_ref, gtb_ref, f_ref, out_ref, d_acc, n_acc, s_acc):
    b = pl.program_id(0)
    cc = pl.program_id(1)

    @pl.when(cc == 0)
    def _reset_image_acc():
        d_acc[...] = jnp.zeros_like(d_acc)
        n_acc[...] = jnp.zeros_like(n_acc)

    @pl.when(jnp.logical_and(b == 0, cc == 0))
    def _reset_global_acc():
        s_acc[0] = jnp.float32(0.0)
        s_acc[1] = jnp.float32(0.0)

    f = f_ref[0]
    n_acc[...] += jnp.sum(f * f, axis=0)
    row_rolled = {0: f}
    for di in (1, 2):
        row_rolled[di] = jnp.roll(f, -di, axis=1)
    for s_idx, (di, dj) in enumerate(_SHIFTS):
        fr = row_rolled[di]
        if dj:
            fr = jnp.roll(fr, -dj, axis=2)
        d_acc[s_idx] += jnp.sum(f * fr, axis=0)

    @pl.when(cc == _NUM_CC - 1)
    def _finalize_image():
        seg = seg_ref[0]
        gtb = gtb_ref[0]
        seg = jnp.where(seg == 255, 0, seg)
        gtb = jnp.where(gtb == 255, 0, gtb)
        r = jax.lax.broadcasted_iota(jnp.int32, (_H, _W), 0)
        c = jax.lax.broadcasted_iota(jnp.int32, (_H, _W), 1)
        interior = (r >= 2) & (r <= _H - 3) & (c >= 2) & (c <= _W - 3)
        maskf = jnp.where((gtb * seg > 0) & interior,
                          jnp.float32(1.0), jnp.float32(0.0))
        inv = jnp.float32(1.0) / jnp.maximum(jnp.sqrt(n_acc[...]),
                                             jnp.float32(_EPS))
        s_total = jnp.float32(0.0)
        for s_idx, (di, dj) in enumerate(_SHIFTS):
            inv_nb = _roll2(inv, di, dj)
            sim = d_acc[s_idx] * inv * inv_nb
            seg_nb = _roll2(seg, di, dj)
            slab = jnp.where(seg == seg_nb, jnp.float32(1.0), jnp.float32(0.0))
            w = maskf + _roll2(maskf, di, dj)
            s_total += jnp.sum(w * (sim - slab) ** 2)
        count = jnp.sum(maskf)
        valid = count >= jnp.float32(1.0)
        contrib = jnp.where(
            valid,
            s_total / (jnp.maximum(count, jnp.float32(1.0)) * jnp.float32(24.0)),
            jnp.float32(0.0))
        s_acc[0] += contrib
        s_acc[1] += jnp.where(valid, jnp.float32(1.0), jnp.float32(0.0))

        @pl.when(b == _B - 1)
        def _finalize_total():
            tot = s_acc[0] / jnp.maximum(s_acc[1], jnp.float32(1.0))
            tot = jnp.where(s_acc[1] == jnp.float32(0.0), jnp.float32(0.0), tot)
            tot = jnp.where(jnp.isnan(tot), jnp.float32(0.0), tot)
            out_ref[...] = jnp.full((1, 1), tot, dtype=jnp.float32)


def kernel(er_input, seg_label, gt_boundary_seg):
    seg_ds = seg_label[:, ::4, ::4]
    gtb_ds = gt_boundary_seg[:, ::4, ::4]

    out = pl.pallas_call(
        _cbl_kernel,
        grid=(_B, _NUM_CC),
        in_specs=[
            pl.BlockSpec((1, _H, _W), lambda b, cc: (b, 0, 0)),
            pl.BlockSpec((1, _H, _W), lambda b, cc: (b, 0, 0)),
            pl.BlockSpec((1, _C_CHUNK, _H, _W), lambda b, cc: (b, cc, 0, 0)),
        ],
        out_specs=pl.BlockSpec((1, 1), lambda b, cc: (0, 0)),
        out_shape=jax.ShapeDtypeStruct((1, 1), jnp.float32),
        scratch_shapes=[
            pltpu.VMEM((len(_SHIFTS), _H, _W), jnp.float32),
            pltpu.VMEM((_H, _W), jnp.float32),
            pltpu.SMEM((2,), jnp.float32),
        ],
    )(seg_ds, gtb_ds, er_input)
    return out.reshape(())


# speedup vs baseline: 16.8254x; 1.1594x over previous
import jax
import jax.numpy as jnp
from jax.experimental import pallas as pl
from jax.experimental.pallas import tpu as pltpu

_EPS = 1e-8
_H = 128
_W = 128
_C = 256
_B = 4
_C_CHUNK = 64
_NUM_CC = _C // _C_CHUNK
_TILE_R = 32
_NUM_T = _H // _TILE_R

_SHIFTS = [(1, 0), (2, 0)] + [(di, dj) for dj in (1, 2) for di in range(-2, 3)]


def _roll2(x, di, dj):
    if di:
        x = jnp.roll(x, -di, axis=x.ndim - 2)
    if dj:
        x = jnp.roll(x, -dj, axis=x.ndim - 1)
    return x


def _cbl_kernel(seg_ref, gtb_ref, f_ref, out_ref, d_acc, n_acc, pad0, pad1,
                pad2, s_acc):
    b = pl.program_id(0)
    cc = pl.program_id(1)

    @pl.when(cc == 0)
    def _reset_image_acc():
        d_acc[...] = jnp.zeros_like(d_acc)
        n_acc[...] = jnp.zeros_like(n_acc)

    @pl.when(jnp.logical_and(b == 0, cc == 0))
    def _reset_global_acc():
        s_acc[0] = jnp.float32(0.0)
        s_acc[1] = jnp.float32(0.0)

    f = f_ref[0]
    zrows = jnp.zeros((_C_CHUNK, 2, _W), jnp.float32)
    for ref, dj in ((pad0, 0), (pad1, 1), (pad2, 2)):
        ref[:, 2:_H + 2, :] = jnp.roll(f, -dj, axis=2) if dj else f
        ref[:, 0:2, :] = zrows
        ref[:, _H + 2:_H + 4, :] = zrows

    pads = (pad0, pad1, pad2)
    for t in range(_NUM_T):
        base = t * _TILE_R

        def body(c, carry):
            accs, nacc = carry
            left = pad0[c, base + 2:base + 2 + _TILE_R, :]
            nacc = nacc + left * left
            new = []
            for s_idx, (di, dj) in enumerate(_SHIFTS):
                right = pads[dj][c, base + 2 + di:base + 2 + di + _TILE_R, :]
                new.append(accs[s_idx] + left * right)
            return tuple(new), nacc

        zero_tile = jnp.zeros((_TILE_R, _W), jnp.float32)
        accs, nacc = jax.lax.fori_loop(
            0, _C_CHUNK, body,
            (tuple(zero_tile for _ in _SHIFTS), zero_tile))
        n_acc[base:base + _TILE_R, :] += nacc
        for s_idx in range(len(_SHIFTS)):
            d_acc[s_idx, base:base + _TILE_R, :] += accs[s_idx]

    @pl.when(cc == _NUM_CC - 1)
    def _finalize_image():
        seg = seg_ref[0]
        gtb = gtb_ref[0]
        seg = jnp.where(seg == 255, 0, seg)
        gtb = jnp.where(gtb == 255, 0, gtb)
        r = jax.lax.broadcasted_iota(jnp.int32, (_H, _W), 0)
        c = jax.lax.broadcasted_iota(jnp.int32, (_H, _W), 1)
        interior = (r >= 2) & (r <= _H - 3) & (c >= 2) & (c <= _W - 3)
        maskf = jnp.where((gtb * seg > 0) & interior,
                          jnp.float32(1.0), jnp.float32(0.0))
        inv = jnp.float32(1.0) / jnp.maximum(jnp.sqrt(n_acc[...]),
                                             jnp.float32(_EPS))
        s_total = jnp.float32(0.0)
        for s_idx, (di, dj) in enumerate(_SHIFTS):
            inv_nb = _roll2(inv, di, dj)
            sim = d_acc[s_idx] * inv * inv_nb
            seg_nb = _roll2(seg, di, dj)
            slab = jnp.where(seg == seg_nb, jnp.float32(1.0), jnp.float32(0.0))
            w = maskf + _roll2(maskf, di, dj)
            s_total += jnp.sum(w * (sim - slab) ** 2)
        count = jnp.sum(maskf)
        valid = count >= jnp.float32(1.0)
        contrib = jnp.where(
            valid,
            s_total / (jnp.maximum(count, jnp.float32(1.0)) * jnp.float32(24.0)),
            jnp.float32(0.0))
        s_acc[0] += contrib
        s_acc[1] += jnp.where(valid, jnp.float32(1.0), jnp.float32(0.0))

        @pl.when(b == _B - 1)
        def _finalize_total():
            tot = s_acc[0] / jnp.maximum(s_acc[1], jnp.float32(1.0))
            tot = jnp.where(s_acc[1] == jnp.float32(0.0), jnp.float32(0.0), tot)
            tot = jnp.where(jnp.isnan(tot), jnp.float32(0.0), tot)
            out_ref[...] = jnp.full((1, 1), tot, dtype=jnp.float32)


def kernel(er_input, seg_label, gt_boundary_seg):
    seg_ds = seg_label[:, ::4, ::4]
    gtb_ds = gt_boundary_seg[:, ::4, ::4]

    out = pl.pallas_call(
        _cbl_kernel,
        grid=(_B, _NUM_CC),
        in_specs=[
            pl.BlockSpec((1, _H, _W), lambda b, cc: (b, 0, 0)),
            pl.BlockSpec((1, _H, _W), lambda b, cc: (b, 0, 0)),
            pl.BlockSpec((1, _C_CHUNK, _H, _W), lambda b, cc: (b, cc, 0, 0)),
        ],
        out_specs=pl.BlockSpec((1, 1), lambda b, cc: (0, 0)),
        out_shape=jax.ShapeDtypeStruct((1, 1), jnp.float32),
        scratch_shapes=[
            pltpu.VMEM((len(_SHIFTS), _H, _W), jnp.float32),
            pltpu.VMEM((_H, _W), jnp.float32),
            pltpu.VMEM((_C_CHUNK, _H + 4, _W), jnp.float32),
            pltpu.VMEM((_C_CHUNK, _H + 4, _W), jnp.float32),
            pltpu.VMEM((_C_CHUNK, _H + 4, _W), jnp.float32),
            pltpu.SMEM((2,), jnp.float32),
        ],
    )(seg_ds, gtb_ds, er_input)
    return out.reshape(())
